# Initial kernel scaffold; baseline (speedup 1.0000x reference)
import jax
import jax.numpy as jnp
from jax.experimental import pallas as pl


def kernel(x, table):
    raise NotImplementedError("write your pallas kernel here")


# speedup vs baseline: 1.0935x; 1.0935x over previous
import functools

import jax
import jax.numpy as jnp
from jax import lax
from jax.experimental import pallas as pl
from jax.experimental.pallas import tpu as pltpu
from jax.experimental.pallas import tpu_sc as plsc

_V = 1000000
_D = 32
_B = 16384 * 50

_NC = 2
_NS = 16
_NW = _NC * _NS
_BPW = _B // _NW
_CHUNK = 1024
_NCH = _BPW // _CHUNK


_mesh = plsc.VectorSubcoreMesh(core_axis_name="c", subcore_axis_name="s")


@functools.partial(
    pl.kernel,
    mesh=_mesh,
    out_type=jax.ShapeDtypeStruct((_B, _D), jnp.float32),
    scratch_types=[
        pltpu.VMEM((_CHUNK,), jnp.int32),
        pltpu.VMEM((_CHUNK, _D), jnp.float32),
        pltpu.SemaphoreType.DMA,
    ],
    compiler_params=pltpu.CompilerParams(use_tc_tiling_on_sc=False),
)
def _emb_lookup(x_hbm, table_hbm, out_hbm, idx_v, rows_v, sem):
    wid = lax.axis_index("s") * _NC + lax.axis_index("c")
    base = wid * _BPW

    def body(i, carry):
        off = base + i * _CHUNK
        pltpu.sync_copy(x_hbm.at[pl.ds(off, _CHUNK)], idx_v)
        pltpu.async_copy(table_hbm.at[idx_v], rows_v, sem).wait()
        pltpu.sync_copy(rows_v, out_hbm.at[pl.ds(off, _CHUNK)])
        return carry

    lax.fori_loop(0, _NCH, body, 0)


def kernel(x, table):
    flat_idx = x.reshape(-1).astype(jnp.int32)
    out = _emb_lookup(flat_idx, table)
    return out.reshape(x.shape[0], x.shape[1], _D)


# speedup vs baseline: 1.1126x; 1.0174x over previous
import functools

import jax
import jax.numpy as jnp
from jax import lax
from jax.experimental import pallas as pl
from jax.experimental.pallas import tpu as pltpu
from jax.experimental.pallas import tpu_sc as plsc

_V = 1000000
_D = 32
_B = 16384 * 50

_NC = 2
_NS = 16
_NW = _NC * _NS
_BPW = _B // _NW
_CHUNK = 800
_NCH = _BPW // _CHUNK
_NBUF = 4
_LOOK = _NBUF - 1


_mesh = plsc.VectorSubcoreMesh(core_axis_name="c", subcore_axis_name="s")


@functools.partial(
    pl.kernel,
    mesh=_mesh,
    out_type=jax.ShapeDtypeStruct((_B, _D), jnp.float32),
    scratch_types=[
        pltpu.VMEM((_BPW,), jnp.int32),
        pltpu.VMEM((_NBUF, _CHUNK, _D), jnp.float32),
        pltpu.SemaphoreType.DMA((_NBUF,)),
        pltpu.SemaphoreType.DMA((_NBUF,)),
    ],
    compiler_params=pltpu.CompilerParams(use_tc_tiling_on_sc=False),
)
def _emb_lookup(x_hbm, table_hbm, out_hbm, idx_v, rows_v, gsem, ssem):
    wid = lax.axis_index("s") * _NC + lax.axis_index("c")
    base = wid * _BPW

    pltpu.sync_copy(x_hbm.at[pl.ds(base, _BPW)], idx_v)

    def gather(i, b):
        return pltpu.make_async_copy(
            table_hbm.at[idx_v.at[pl.ds(i * _CHUNK, _CHUNK)]],
            rows_v.at[b],
            gsem.at[b],
        )

    def store(i, b):
        return pltpu.make_async_copy(
            rows_v.at[b],
            out_hbm.at[pl.ds(base + i * _CHUNK, _CHUNK)],
            ssem.at[b],
        )

    for b in range(_LOOK):
        gather(b, b).start()

    def outer(g, carry):
        for b in range(_NBUF):
            i = g * _NBUF + b
            gather(i, b).wait()

            bp = (b - 1) % _NBUF

            @pl.when(i >= 1)
            def _():
                store(i - 1, bp).wait()

            bn = (b + _LOOK) % _NBUF

            @pl.when(i + _LOOK < _NCH)
            def _():
                gather(i + _LOOK, bn).start()

            store(i, b).start()
        return carry

    lax.fori_loop(0, _NCH // _NBUF, outer, 0)
    store(_NCH - 1, (_NCH - 1) % _NBUF).wait()


def kernel(x, table):
    flat_idx = x.reshape(-1).astype(jnp.int32)
    out = _emb_lookup(flat_idx, table)
    return out.reshape(x.shape[0], x.shape[1], _D)


# speedup vs baseline: 1.9418x; 1.7453x over previous
import functools

import jax
import jax.numpy as jnp
from jax import lax
from jax.experimental import pallas as pl
from jax.experimental.pallas import tpu as pltpu
from jax.experimental.pallas import tpu_sc as plsc

_V = 1000000
_D = 32
_B = 16384 * 50

_NC = 2
_NS = 16
_NW = _NC * _NS
_BPW = _B // _NW
_CHUNK = 800
_NCH = _BPW // _CHUNK
_NBUF = 4
_LOOK = _NBUF - 1


_mesh = plsc.VectorSubcoreMesh(core_axis_name="c", subcore_axis_name="s")


@functools.partial(
    pl.kernel,
    mesh=_mesh,
    out_type=jax.ShapeDtypeStruct((_B, _D), jnp.float32),
    scratch_types=[
        pltpu.VMEM((_BPW,), jnp.int32),
        pltpu.VMEM((_NBUF, _CHUNK, _D), jnp.float32),
        pltpu.SemaphoreType.DMA((_NBUF,)),
        pltpu.SemaphoreType.DMA((_NBUF,)),
    ],
    compiler_params=pltpu.CompilerParams(use_tc_tiling_on_sc=False),
)
def _emb_lookup(x_hbm, table_hbm, out_hbm, idx_v, rows_v, gsem, ssem):
    wid = lax.axis_index("s") * _NC + lax.axis_index("c")
    base = wid * _BPW

    pltpu.sync_copy(x_hbm.at[pl.ds(base, _BPW)], idx_v)

    def gather(i, b):
        return pltpu.make_async_copy(
            table_hbm.at[idx_v.at[pl.ds(i * _CHUNK, _CHUNK)]],
            rows_v.at[b],
            gsem.at[b],
        )

    def store(i, b):
        return pltpu.make_async_copy(
            rows_v.at[b],
            out_hbm.at[pl.ds(base + i * _CHUNK, _CHUNK)],
            ssem.at[b],
        )

    for b in range(_LOOK):
        gather(b, b).start()

    def outer(g, carry):
        for b in range(_NBUF):
            i = g * _NBUF + b
            gather(i, b).wait()

            bp = (b - 1) % _NBUF

            @pl.when(i >= 1)
            def _():
                store(i - 1, bp).wait()

            bn = (b + _LOOK) % _NBUF

            @pl.when(i + _LOOK < _NCH)
            def _():
                gather(i + _LOOK, bn).start()

            store(i, b).start()
        return carry

    lax.fori_loop(0, _NCH // _NBUF, outer, 0)
    store(_NCH - 1, (_NCH - 1) % _NBUF).wait()


def kernel(x, table):
    flat_idx = jnp.transpose(x).reshape(-1).astype(jnp.int32)
    out = _emb_lookup(flat_idx, table)
    return out.reshape(x.shape[1], x.shape[0], _D).transpose(1, 0, 2)
